# Initial kernel scaffold; baseline (speedup 1.0000x reference)
import jax
import jax.numpy as jnp
from jax.experimental import pallas as pl


def kernel(inputs, dictionary):
    raise NotImplementedError("write your pallas kernel here")


# speedup vs baseline: 2.1999x; 2.1999x over previous
import functools

import jax
import jax.numpy as jnp
from jax import lax
from jax.experimental import pallas as pl


def _vq_body(dict_ref, z_ref, emb_ref, emb_pt_ref, idx_ref):
    d = dict_ref[...]
    z = z_ref[0]
    dots = lax.dot_general(d, z, (((1,), (0,)), ((), ())),
                           preferred_element_type=jnp.float32)
    dn = jnp.sum(d * d, axis=1)
    zn = jnp.sum(z * z, axis=0)
    dist = (-2.0 * dots + dn[:, None]) + zn[None, :]
    m = jnp.min(dist, axis=0)
    iota = lax.broadcasted_iota(jnp.int32, (1024, 1024), 0)
    big = jnp.int32(2 ** 30)
    idx = jnp.min(jnp.where(dist == m[None, :], iota, big), axis=0)
    onehot = (iota == idx[None, :]).astype(jnp.float32)
    emb = lax.dot_general(d, onehot, (((0,), (0,)), ((), ())),
                          preferred_element_type=jnp.float32)
    emb_ref[0] = emb
    emb_pt_ref[0] = emb
    idx_ref[0] = idx.reshape(8, 128)


@jax.jit
def kernel(inputs, dictionary):
    n, c, h, w = inputs.shape
    t = h * w
    z = inputs.reshape(n, c, t)
    emb, emb_pt, idx = pl.pallas_call(
        _vq_body,
        grid=(n,),
        in_specs=[
            pl.BlockSpec((1024, 64), lambda i: (0, 0)),
            pl.BlockSpec((1, c, t), lambda i: (i, 0, 0)),
        ],
        out_specs=[
            pl.BlockSpec((1, c, t), lambda i: (i, 0, 0)),
            pl.BlockSpec((1, c, t), lambda i: (i, 0, 0)),
            pl.BlockSpec((1, 8, 128), lambda i: (i, 0, 0)),
        ],
        out_shape=[
            jax.ShapeDtypeStruct((n, c, t), jnp.float32),
            jax.ShapeDtypeStruct((n, c, t), jnp.float32),
            jax.ShapeDtypeStruct((n, 8, 128), jnp.int32),
        ],
    )(dictionary, z)
    emb = emb.reshape(n, c, h, w)
    emb_pt = emb_pt.reshape(n, c, h, w)
    idx = idx.reshape(n, h, w)
    return emb, emb_pt, idx


# speedup vs baseline: 2.5891x; 1.1769x over previous
import jax
import jax.numpy as jnp
from jax import lax
from jax.experimental import pallas as pl
from jax.experimental.pallas import tpu as pltpu

_TB = 1024


def _vq_body(dict_ref, z_ref, emb_ref, emb_pt_ref, idx_ref, d2_ref, dn_ref):
    @pl.when(pl.program_id(0) == 0)
    def _prologue():
        d = dict_ref[...]
        d2_ref[...] = -2.0 * d
        dn_ref[...] = jnp.sum(d * d, axis=1, keepdims=True)

    d2 = d2_ref[...]
    z = z_ref[0]
    dots2 = lax.dot_general(d2, z, (((1,), (0,)), ((), ())),
                            preferred_element_type=jnp.float32)
    zn = jnp.sum(z * z, axis=0)
    dist = (dots2 + dn_ref[...]) + zn[None, :]
    idx = jnp.argmin(dist, axis=0).astype(jnp.int32)
    iota = lax.broadcasted_iota(jnp.int32, (1024, _TB), 0)
    onehot = (iota == idx[None, :]).astype(jnp.float32)
    emb = lax.dot_general(dict_ref[...], onehot, (((0,), (0,)), ((), ())),
                          preferred_element_type=jnp.float32)
    emb_ref[0] = emb
    emb_pt_ref[0] = emb
    idx_ref[0] = idx.reshape(_TB // 128, 128)


@jax.jit
def kernel(inputs, dictionary):
    n, c, h, w = inputs.shape
    t = h * w
    z = inputs.reshape(n, c, t)
    emb, emb_pt, idx = pl.pallas_call(
        _vq_body,
        grid=(n,),
        in_specs=[
            pl.BlockSpec((1024, 64), lambda i: (0, 0)),
            pl.BlockSpec((1, c, t), lambda i: (i, 0, 0)),
        ],
        out_specs=[
            pl.BlockSpec((1, c, t), lambda i: (i, 0, 0)),
            pl.BlockSpec((1, c, t), lambda i: (i, 0, 0)),
            pl.BlockSpec((1, 8, 128), lambda i: (i, 0, 0)),
        ],
        out_shape=[
            jax.ShapeDtypeStruct((n, c, t), jnp.float32),
            jax.ShapeDtypeStruct((n, c, t), jnp.float32),
            jax.ShapeDtypeStruct((n, 8, 128), jnp.int32),
        ],
        scratch_shapes=[
            pltpu.VMEM((1024, 64), jnp.float32),
            pltpu.VMEM((1024, 1), jnp.float32),
        ],
    )(dictionary, z)
    emb = emb.reshape(n, c, h, w)
    emb_pt = emb_pt.reshape(n, c, h, w)
    idx = idx.reshape(n, h, w)
    return emb, emb_pt, idx
